# baseline (speedup 1.0000x reference)
import functools

import jax
import jax.numpy as jnp
from jax import lax
from jax.experimental import pallas as pl
from jax.experimental.pallas import tpu as pltpu
from jax.experimental.pallas import tpu_sc as plsc

NC, NS, L = 2, 16, 16
NW = NC * NS
B, C, T = 1024, 1000, 10
NTILES = C // 8
TPW = 31
NSB = 128 // L

_mesh = plsc.VectorSubcoreMesh(
    core_axis_name="c", subcore_axis_name="s", num_cores=NC, num_subcores=NS
)


@functools.partial(
    pl.kernel,
    out_type=jax.ShapeDtypeStruct((2 * NW, L), jnp.float32),
    mesh=_mesh,
    compiler_params=pltpu.CompilerParams(
        needs_layout_passes=False, use_tc_tiling_on_sc=True
    ),
    scratch_types=[
        pltpu.VMEM((256, 128), jnp.float32),
        pltpu.VMEM((256, 128), jnp.float32),
        pltpu.VMEM((T, 128), jnp.float32),
        pltpu.VMEM((T, 128), jnp.float32),
        pltpu.VMEM((T, 128), jnp.int32),
        pltpu.VMEM((L,), jnp.int32),
        pltpu.VMEM((T, L), jnp.float32),
        pltpu.VMEM((L,), jnp.float32),
        pltpu.VMEM((L,), jnp.float32),
        pltpu.VMEM_SHARED((4 * T, 128), jnp.float32),
        pltpu.SemaphoreType.DMA,
        pltpu.SemaphoreType.DMA,
    ],
)
def _loss_kernel(xt_hbm, mt_hbm, tgt_hbm, out,
                 xa, ma, thb, pth, tbuf, idxb, accv, avec, tvec,
                 shared, sem0, sem1):
    c = lax.axis_index("c")
    s = lax.axis_index("s")
    wid = s * NC + c
    tc = c * 4 + jnp.bitwise_and(s, 3)
    tcl = jnp.bitwise_and(s, 3)
    q = jnp.right_shift(s, 2)
    tbase = TPW * q
    col0 = 128 * tc

    pltpu.async_copy(
        xt_hbm.at[pl.ds(8 * tbase, 8 * TPW), pl.ds(col0, 128)],
        xa.at[pl.ds(0, 8 * TPW)], sem0)
    pltpu.async_copy(
        mt_hbm.at[pl.ds(8 * tbase, 8 * TPW), pl.ds(col0, 128)],
        ma.at[pl.ds(0, 8 * TPW)], sem1)
    t32 = jnp.where(q == 0, NTILES - 1, tbase)
    pltpu.async_copy(
        xt_hbm.at[pl.ds(8 * t32, 8), pl.ds(col0, 128)],
        xa.at[pl.ds(8 * TPW, 8)], sem0)
    pltpu.async_copy(
        mt_hbm.at[pl.ds(8 * t32, 8), pl.ds(col0, 128)],
        ma.at[pl.ds(8 * TPW, 8)], sem1)

    pltpu.sync_copy(tgt_hbm.at[pl.ds(0, 8), pl.ds(col0, 128)], tbuf.at[pl.ds(0, 8)])
    pltpu.sync_copy(tgt_hbm.at[pl.ds(8, 2), pl.ds(col0, 128)], tbuf.at[pl.ds(8, 2)])

    zero = jnp.zeros((L,), jnp.float32)

    def zero_body(t, carry):
        for sb in range(NSB):
            thb[t, pl.ds(L * sb, L)] = zero
        return carry

    lax.fori_loop(0, T, zero_body, jnp.int32(0))

    lanes = lax.iota(jnp.int32, L)
    idxb[...] = lanes + T * tcl

    def tgt_addr(t, sb):
        tg = tbuf[t, pl.ds(L * sb, L)]
        tr = jnp.right_shift(tg, 3)
        tl = tr - tbase
        extra = jnp.logical_and(tr == NTILES - 1, q == 0)
        valid = jnp.logical_or(jnp.logical_and(tl >= 0, tl < TPW), extra)
        idx = jnp.clip(jnp.where(extra, TPW, tl), 0, TPW)
        row = jnp.left_shift(idx, 3) + jnp.bitwise_and(tg, 7)
        return row, lanes + L * sb, valid

    pltpu.make_async_copy(
        xt_hbm.at[pl.ds(0, 8 * TPW), pl.ds(0, 128)],
        xa.at[pl.ds(0, 8 * TPW)], sem0).wait()
    pltpu.make_async_copy(
        xt_hbm.at[pl.ds(0, 8), pl.ds(0, 128)], xa.at[pl.ds(0, 8)], sem0).wait()

    def xg_body(i, carry):
        t, sb = jnp.right_shift(i, 3), jnp.bitwise_and(i, NSB - 1)
        row, col, _ = tgt_addr(t, sb)
        pth[t, pl.ds(L * sb, L)] = plsc.load_gather(xa, [row, col])
        return carry

    lax.fori_loop(0, T * NSB, xg_body, jnp.int32(0))

    pltpu.make_async_copy(
        mt_hbm.at[pl.ds(0, 8 * TPW), pl.ds(0, 128)],
        ma.at[pl.ds(0, 8 * TPW)], sem1).wait()
    pltpu.make_async_copy(
        mt_hbm.at[pl.ds(0, 8), pl.ds(0, 128)], ma.at[pl.ds(0, 8)], sem1).wait()

    def mg_body(i, psum):
        t, sb = jnp.right_shift(i, 3), jnp.bitwise_and(i, NSB - 1)
        row, col, valid = tgt_addr(t, sb)
        g = pth[t, pl.ds(L * sb, L)] - plsc.load_gather(ma, [row, col])
        psum = psum + jnp.where(valid, g, 0.0)
        pth[t, pl.ds(L * sb, L)] = jnp.where(valid, g - 1.0, 0.0)
        return psum

    psum = lax.fori_loop(0, T * NSB, mg_body, zero)

    @pl.when(q == 0)
    def _():
        pltpu.sync_copy(thb, shared.at[pl.ds(T * tcl, T)])
    plsc.subcore_barrier()
    pltpu.sync_copy(pth, shared.at[idxb.at[pl.ds(0, T)]], add=True)
    plsc.subcore_barrier()
    pltpu.sync_copy(shared.at[pl.ds(T * tcl, T)], thb)

    def tile_body(i, accs):
        base = jnp.left_shift(i, 3)

        def sb_body(sb, accs):
            accs = list(accs)
            ths = [thb[t, pl.ds(L * sb, L)] for t in range(T)]
            for r in range(8):
                s_ = xa[base + r, pl.ds(L * sb, L)] - ma[base + r, pl.ds(L * sb, L)]
                for t in range(T):
                    accs[t] = accs[t] + jnp.maximum(s_, ths[t])
            return tuple(accs)

        return lax.fori_loop(0, NSB, sb_body, accs)

    ntiles = jnp.where(q == 0, TPW + 1, TPW)
    accs = lax.fori_loop(0, ntiles, tile_body, tuple(zero for _ in range(T)))

    acc = accs[0]
    for t in range(1, T):
        acc = acc + accs[t]
    avec[...] = acc
    tvec[...] = psum * jnp.float32(-C)
    pltpu.sync_copy(avec, out.at[wid])
    pltpu.sync_copy(tvec, out.at[NW + wid])


def kernel(input_data, target, adaptive_margin):
    out = _loss_kernel(
        input_data.T, adaptive_margin.T, target.astype(jnp.int32).T
    )
    total = jnp.sum(out) + jnp.float32((C - 1) * B * T)
    return total / jnp.float32(C)


# speedup vs baseline: 1.0301x; 1.0301x over previous
import functools

import jax
import jax.numpy as jnp
from jax import lax
from jax.experimental import pallas as pl
from jax.experimental.pallas import tpu as pltpu
from jax.experimental.pallas import tpu_sc as plsc

NC, NS, L = 2, 16, 16
NW = NC * NS
B, C, T = 1024, 1000, 10
NTILES = C // 8
TPW = 31
NSB = 128 // L

_mesh = plsc.VectorSubcoreMesh(
    core_axis_name="c", subcore_axis_name="s", num_cores=NC, num_subcores=NS
)


@functools.partial(
    pl.kernel,
    out_type=jax.ShapeDtypeStruct((2 * NW, L), jnp.float32),
    mesh=_mesh,
    compiler_params=pltpu.CompilerParams(
        needs_layout_passes=False, use_tc_tiling_on_sc=True
    ),
    scratch_types=[
        pltpu.VMEM((256, 128), jnp.float32),
        pltpu.VMEM((256, 128), jnp.float32),
        pltpu.VMEM((T, 128), jnp.float32),
        pltpu.VMEM((T, 128), jnp.float32),
        pltpu.VMEM((T, 128), jnp.int32),
        pltpu.VMEM((L,), jnp.int32),
        pltpu.VMEM((T, L), jnp.float32),
        pltpu.VMEM((L,), jnp.float32),
        pltpu.VMEM((L,), jnp.float32),
        pltpu.VMEM_SHARED((4 * T, 128), jnp.float32),
        pltpu.SemaphoreType.DMA,
        pltpu.SemaphoreType.DMA,
    ],
)
def _loss_kernel(xt_hbm, mt_hbm, tgt_hbm, out,
                 xa, ma, thb, pth, tbuf, idxb, accv, avec, tvec,
                 shared, sem0, sem1):
    c = lax.axis_index("c")
    s = lax.axis_index("s")
    wid = s * NC + c
    tc = c * 4 + jnp.bitwise_and(s, 3)
    tcl = jnp.bitwise_and(s, 3)
    q = jnp.right_shift(s, 2)
    tbase = TPW * q
    col0 = 128 * tc

    pltpu.async_copy(
        xt_hbm.at[pl.ds(8 * tbase, 8 * TPW), pl.ds(col0, 128)],
        xa.at[pl.ds(0, 8 * TPW)], sem0)
    pltpu.async_copy(
        mt_hbm.at[pl.ds(8 * tbase, 8 * TPW), pl.ds(col0, 128)],
        ma.at[pl.ds(0, 8 * TPW)], sem1)
    t32 = jnp.where(q == 0, NTILES - 1, tbase)
    pltpu.async_copy(
        xt_hbm.at[pl.ds(8 * t32, 8), pl.ds(col0, 128)],
        xa.at[pl.ds(8 * TPW, 8)], sem0)
    pltpu.async_copy(
        mt_hbm.at[pl.ds(8 * t32, 8), pl.ds(col0, 128)],
        ma.at[pl.ds(8 * TPW, 8)], sem1)

    pltpu.sync_copy(tgt_hbm.at[pl.ds(0, 8), pl.ds(col0, 128)], tbuf.at[pl.ds(0, 8)])
    pltpu.sync_copy(tgt_hbm.at[pl.ds(8, 2), pl.ds(col0, 128)], tbuf.at[pl.ds(8, 2)])

    zero = jnp.zeros((L,), jnp.float32)

    def zero_body(t, carry):
        for sb in range(NSB):
            thb[t, pl.ds(L * sb, L)] = zero
        return carry

    lax.fori_loop(0, T, zero_body, jnp.int32(0))

    lanes = lax.iota(jnp.int32, L)
    idxb[...] = lanes + T * tcl

    def tgt_addr(t, sb):
        tg = tbuf[t, pl.ds(L * sb, L)]
        tr = jnp.right_shift(tg, 3)
        tl = tr - tbase
        extra = jnp.logical_and(tr == NTILES - 1, q == 0)
        valid = jnp.logical_or(jnp.logical_and(tl >= 0, tl < TPW), extra)
        idx = jnp.clip(jnp.where(extra, TPW, tl), 0, TPW)
        row = jnp.left_shift(idx, 3) + jnp.bitwise_and(tg, 7)
        return row, lanes + L * sb, valid

    pltpu.make_async_copy(
        xt_hbm.at[pl.ds(0, 8 * TPW), pl.ds(0, 128)],
        xa.at[pl.ds(0, 8 * TPW)], sem0).wait()
    pltpu.make_async_copy(
        xt_hbm.at[pl.ds(0, 8), pl.ds(0, 128)], xa.at[pl.ds(0, 8)], sem0).wait()

    def xg_body(t, carry):
        for sb in range(NSB):
            row, col, _ = tgt_addr(t, sb)
            pth[t, pl.ds(L * sb, L)] = plsc.load_gather(xa, [row, col])
        return carry

    lax.fori_loop(0, T, xg_body, jnp.int32(0))

    pltpu.make_async_copy(
        mt_hbm.at[pl.ds(0, 8 * TPW), pl.ds(0, 128)],
        ma.at[pl.ds(0, 8 * TPW)], sem1).wait()
    pltpu.make_async_copy(
        mt_hbm.at[pl.ds(0, 8), pl.ds(0, 128)], ma.at[pl.ds(0, 8)], sem1).wait()

    def mg_body(t, psum):
        for sb in range(NSB):
            row, col, valid = tgt_addr(t, sb)
            g = pth[t, pl.ds(L * sb, L)] - plsc.load_gather(ma, [row, col])
            psum = psum + jnp.where(valid, g, 0.0)
            pth[t, pl.ds(L * sb, L)] = jnp.where(valid, g - 1.0, 0.0)
        return psum

    psum = lax.fori_loop(0, T, mg_body, zero)

    @pl.when(q == 0)
    def _():
        pltpu.sync_copy(thb, shared.at[pl.ds(T * tcl, T)])
    plsc.subcore_barrier()
    pltpu.sync_copy(pth, shared.at[idxb.at[pl.ds(0, T)]], add=True)
    plsc.subcore_barrier()
    pltpu.sync_copy(shared.at[pl.ds(T * tcl, T)], thb)

    def tile_body(i, accs):
        base = jnp.left_shift(i, 3)

        def sb_body(sb, accs):
            accs = list(accs)
            ths = [thb[t, pl.ds(L * sb, L)] for t in range(T)]
            for r in range(8):
                s_ = xa[base + r, pl.ds(L * sb, L)] - ma[base + r, pl.ds(L * sb, L)]
                for t in range(T):
                    accs[t] = accs[t] + jnp.maximum(s_, ths[t])
            return tuple(accs)

        return lax.fori_loop(0, NSB, sb_body, accs)

    ntiles = jnp.where(q == 0, TPW + 1, TPW)
    accs = lax.fori_loop(0, ntiles, tile_body, tuple(zero for _ in range(T)))

    acc = accs[0]
    for t in range(1, T):
        acc = acc + accs[t]
    avec[...] = acc
    tvec[...] = psum * jnp.float32(-C)
    pltpu.sync_copy(avec, out.at[wid])
    pltpu.sync_copy(tvec, out.at[NW + wid])


def kernel(input_data, target, adaptive_margin):
    out = _loss_kernel(
        input_data.T, adaptive_margin.T, target.astype(jnp.int32).T
    )
    total = jnp.sum(out) + jnp.float32((C - 1) * B * T)
    return total / jnp.float32(C)


# speedup vs baseline: 1.0961x; 1.0640x over previous
import functools

import jax
import jax.numpy as jnp
from jax import lax
from jax.experimental import pallas as pl
from jax.experimental.pallas import tpu as pltpu
from jax.experimental.pallas import tpu_sc as plsc

NC, NS, L = 2, 16, 16
NW = NC * NS
B, C, T = 1024, 1000, 10
NTILES = C // 8
TPW = 25
NSB = 128 // L
SC_BLOCKS = 6

_mesh = plsc.VectorSubcoreMesh(
    core_axis_name="c", subcore_axis_name="s", num_cores=NC, num_subcores=NS
)


@functools.partial(
    pl.kernel,
    out_type=jax.ShapeDtypeStruct((2 * NW, L), jnp.float32),
    mesh=_mesh,
    compiler_params=pltpu.CompilerParams(
        needs_layout_passes=False, use_tc_tiling_on_sc=True
    ),
    scratch_types=[
        pltpu.VMEM((8 * TPW, 128), jnp.float32),
        pltpu.VMEM((8 * TPW, 128), jnp.float32),
        pltpu.VMEM((T, 128), jnp.float32),
        pltpu.VMEM((T, 128), jnp.float32),
        pltpu.VMEM((T, 128), jnp.int32),
        pltpu.VMEM((L,), jnp.int32),
        pltpu.VMEM((L,), jnp.float32),
        pltpu.VMEM((L,), jnp.float32),
        pltpu.VMEM_SHARED((3 * T, 128), jnp.float32),
        pltpu.SemaphoreType.DMA,
        pltpu.SemaphoreType.DMA,
    ],
)
def _loss_kernel(xt_hbm, mt_hbm, tgt_hbm, out,
                 xa, ma, thb, pth, tbuf, idxb, avec, tvec,
                 shared, sem0, sem1):
    c = lax.axis_index("c")
    s = lax.axis_index("s")
    wid = s * NC + c
    active = s < 15
    blk = jnp.remainder(s, 3)
    tc = c * 3 + blk
    q = jnp.minimum(s // 3, 4)
    tbase = TPW * q
    col0 = 128 * tc

    pltpu.async_copy(
        xt_hbm.at[pl.ds(8 * tbase, 8 * TPW), pl.ds(col0, 128)],
        xa.at[pl.ds(0, 8 * TPW)], sem0)
    pltpu.async_copy(
        mt_hbm.at[pl.ds(8 * tbase, 8 * TPW), pl.ds(col0, 128)],
        ma.at[pl.ds(0, 8 * TPW)], sem1)

    pltpu.sync_copy(tgt_hbm.at[pl.ds(0, 8), pl.ds(col0, 128)], tbuf.at[pl.ds(0, 8)])
    pltpu.sync_copy(tgt_hbm.at[pl.ds(8, 2), pl.ds(col0, 128)], tbuf.at[pl.ds(8, 2)])

    zero = jnp.zeros((L,), jnp.float32)

    def zero_body(t, carry):
        for sb in range(NSB):
            thb[t, pl.ds(L * sb, L)] = zero
        return carry

    lax.fori_loop(0, T, zero_body, jnp.int32(0))

    lanes = lax.iota(jnp.int32, L)
    idxb[...] = lanes + T * blk

    def tgt_addr(t, sb):
        tg = tbuf[t, pl.ds(L * sb, L)]
        tl = jnp.right_shift(tg, 3) - tbase
        valid = jnp.logical_and(
            jnp.logical_and(tl >= 0, tl < TPW), active)
        idx = jnp.clip(tl, 0, TPW - 1)
        row = jnp.left_shift(idx, 3) + jnp.bitwise_and(tg, 7)
        return row, lanes + L * sb, valid

    pltpu.make_async_copy(
        xt_hbm.at[pl.ds(0, 8 * TPW), pl.ds(0, 128)],
        xa.at[pl.ds(0, 8 * TPW)], sem0).wait()

    def xg_body(t, carry):
        for sb in range(NSB):
            row, col, _ = tgt_addr(t, sb)
            pth[t, pl.ds(L * sb, L)] = plsc.load_gather(xa, [row, col])
        return carry

    lax.fori_loop(0, T, xg_body, jnp.int32(0))

    pltpu.make_async_copy(
        mt_hbm.at[pl.ds(0, 8 * TPW), pl.ds(0, 128)],
        ma.at[pl.ds(0, 8 * TPW)], sem1).wait()

    def mg_body(t, psum):
        for sb in range(NSB):
            row, col, valid = tgt_addr(t, sb)
            g = pth[t, pl.ds(L * sb, L)] - plsc.load_gather(ma, [row, col])
            psum = psum + jnp.where(valid, g, 0.0)
            pth[t, pl.ds(L * sb, L)] = jnp.where(valid, g - 1.0, 0.0)
        return psum

    psum = lax.fori_loop(0, T, mg_body, zero)

    @pl.when(jnp.logical_and(q == 0, active))
    def _():
        pltpu.sync_copy(thb, shared.at[pl.ds(T * blk, T)])
    plsc.subcore_barrier()
    pltpu.sync_copy(pth, shared.at[idxb.at[pl.ds(0, T)]], add=True)
    plsc.subcore_barrier()
    pltpu.sync_copy(shared.at[pl.ds(T * blk, T)], thb)

    def tile_body(i, accs):
        base = jnp.left_shift(i, 3)

        def sb_body(sb, accs):
            accs = list(accs)
            ths = [thb[t, pl.ds(L * sb, L)] for t in range(T)]
            for r in range(8):
                s_ = xa[base + r, pl.ds(L * sb, L)] - ma[base + r, pl.ds(L * sb, L)]
                for t in range(T):
                    accs[t] = accs[t] + jnp.maximum(s_, ths[t])
            return tuple(accs)

        return lax.fori_loop(0, NSB, sb_body, accs)

    accs = lax.fori_loop(0, TPW, tile_body, tuple(zero for _ in range(T)))

    acc = accs[0]
    for t in range(1, T):
        acc = acc + accs[t]
    gate = jnp.where(active, 1.0, 0.0).astype(jnp.float32)
    avec[...] = acc * gate
    tvec[...] = psum * jnp.float32(-C)
    pltpu.sync_copy(avec, out.at[wid])
    pltpu.sync_copy(tvec, out.at[NW + wid])


TC_COLS = 128 * (8 - SC_BLOCKS)


def _tc_body(x_ref, m_ref, tgt_ref, o_ref):
    d = x_ref[...] - m_ref[...]
    j = lax.broadcasted_iota(jnp.int32, (C, 1), 0)
    acc = jnp.float32(0.0)
    gsum = jnp.float32(0.0)
    for t in range(T):
        tg = tgt_ref[t, :][None, :]
        g = jnp.sum(jnp.where(j == tg, d, 0.0), axis=0)
        th = g - 1.0
        acc = acc + jnp.sum(jnp.maximum(d, th[None, :]))
        gsum = gsum + jnp.sum(g)
    o_ref[...] = jnp.full((8, 128), acc - jnp.float32(C) * gsum, jnp.float32)


_tc_call = pl.pallas_call(
    _tc_body,
    grid=(1,),
    in_specs=[
        pl.BlockSpec((C, TC_COLS), lambda i: (0, SC_BLOCKS * 128 // TC_COLS)),
        pl.BlockSpec((C, TC_COLS), lambda i: (0, SC_BLOCKS * 128 // TC_COLS)),
        pl.BlockSpec((T, TC_COLS), lambda i: (0, SC_BLOCKS * 128 // TC_COLS)),
    ],
    out_specs=pl.BlockSpec((8, 128), lambda i: (0, 0)),
    out_shape=jax.ShapeDtypeStruct((8, 128), jnp.float32),
)


def kernel(input_data, target, adaptive_margin):
    xt = input_data.T
    mt = adaptive_margin.T
    tgtT = target.astype(jnp.int32).T
    sc_out = _loss_kernel(xt, mt, tgtT)
    tc_out = _tc_call(xt, mt, tgtT)
    total = jnp.sum(sc_out) + tc_out[0, 0] + jnp.float32((C - 1) * B * T)
    return total / jnp.float32(C)


# speedup vs baseline: 1.1951x; 1.0904x over previous
import functools

import jax
import jax.numpy as jnp
from jax import lax
from jax.experimental import pallas as pl
from jax.experimental.pallas import tpu as pltpu
from jax.experimental.pallas import tpu_sc as plsc

NC, NS, L = 2, 16, 16
NW = NC * NS
B, C, T = 1024, 1000, 10
NTILES = C // 8
TPW = 16
NSB = 128 // L
SC_BLOCKS = 4

_mesh = plsc.VectorSubcoreMesh(
    core_axis_name="c", subcore_axis_name="s", num_cores=NC, num_subcores=NS
)


@functools.partial(
    pl.kernel,
    out_type=jax.ShapeDtypeStruct((2 * NW, L), jnp.float32),
    mesh=_mesh,
    compiler_params=pltpu.CompilerParams(
        needs_layout_passes=False, use_tc_tiling_on_sc=True
    ),
    scratch_types=[
        pltpu.VMEM((8 * TPW, 128), jnp.float32),
        pltpu.VMEM((8 * TPW, 128), jnp.float32),
        pltpu.VMEM((T, 128), jnp.float32),
        pltpu.VMEM((T, 128), jnp.float32),
        pltpu.VMEM((T, 128), jnp.int32),
        pltpu.VMEM((L,), jnp.int32),
        pltpu.VMEM((L,), jnp.float32),
        pltpu.VMEM((L,), jnp.float32),
        pltpu.VMEM_SHARED((2 * T, 128), jnp.float32),
        pltpu.SemaphoreType.DMA,
        pltpu.SemaphoreType.DMA,
    ],
)
def _loss_kernel(xt_hbm, mt_hbm, tgt_hbm, out,
                 xa, ma, thb, pth, tbuf, idxb, avec, tvec,
                 shared, sem0, sem1):
    c = lax.axis_index("c")
    s = lax.axis_index("s")
    wid = s * NC + c
    blk = jnp.bitwise_and(s, 1)
    tc = c * 2 + blk
    q = jnp.right_shift(s, 1)
    ntiles = jnp.where(q < 5, 16, 15)
    tbase = jnp.where(q <= 5, 16 * q, 80 + 15 * (q - 5))
    fbase = jnp.minimum(tbase, NTILES - TPW)
    off = tbase - fbase
    col0 = 128 * tc

    pltpu.async_copy(
        xt_hbm.at[pl.ds(8 * fbase, 8 * TPW), pl.ds(col0, 128)],
        xa.at[pl.ds(0, 8 * TPW)], sem0)
    pltpu.async_copy(
        mt_hbm.at[pl.ds(8 * fbase, 8 * TPW), pl.ds(col0, 128)],
        ma.at[pl.ds(0, 8 * TPW)], sem1)

    pltpu.sync_copy(tgt_hbm.at[pl.ds(0, 8), pl.ds(col0, 128)], tbuf.at[pl.ds(0, 8)])
    pltpu.sync_copy(tgt_hbm.at[pl.ds(8, 2), pl.ds(col0, 128)], tbuf.at[pl.ds(8, 2)])

    zero = jnp.zeros((L,), jnp.float32)

    def zero_body(t, carry):
        for sb in range(NSB):
            thb[t, pl.ds(L * sb, L)] = zero
        return carry

    lax.fori_loop(0, T, zero_body, jnp.int32(0))

    lanes = lax.iota(jnp.int32, L)
    idxb[...] = lanes + T * blk

    def tgt_addr(t, sb):
        tg = tbuf[t, pl.ds(L * sb, L)]
        tl = jnp.right_shift(tg, 3) - tbase
        valid = jnp.logical_and(tl >= 0, tl < ntiles)
        idx = jnp.clip(tl + off, 0, TPW - 1)
        row = jnp.left_shift(idx, 3) + jnp.bitwise_and(tg, 7)
        return row, lanes + L * sb, valid

    pltpu.make_async_copy(
        xt_hbm.at[pl.ds(0, 8 * TPW), pl.ds(0, 128)],
        xa.at[pl.ds(0, 8 * TPW)], sem0).wait()

    def xg_body(t, carry):
        for sb in range(NSB):
            row, col, _ = tgt_addr(t, sb)
            pth[t, pl.ds(L * sb, L)] = plsc.load_gather(xa, [row, col])
        return carry

    lax.fori_loop(0, T, xg_body, jnp.int32(0))

    pltpu.make_async_copy(
        mt_hbm.at[pl.ds(0, 8 * TPW), pl.ds(0, 128)],
        ma.at[pl.ds(0, 8 * TPW)], sem1).wait()

    def mg_body(t, psum):
        for sb in range(NSB):
            row, col, valid = tgt_addr(t, sb)
            g = pth[t, pl.ds(L * sb, L)] - plsc.load_gather(ma, [row, col])
            psum = psum + jnp.where(valid, g, 0.0)
            pth[t, pl.ds(L * sb, L)] = jnp.where(valid, g - 1.0, 0.0)
        return psum

    psum = lax.fori_loop(0, T, mg_body, zero)

    @pl.when(q == 0)
    def _():
        pltpu.sync_copy(thb, shared.at[pl.ds(T * blk, T)])
    plsc.subcore_barrier()
    pltpu.sync_copy(pth, shared.at[idxb.at[pl.ds(0, T)]], add=True)
    plsc.subcore_barrier()
    pltpu.sync_copy(shared.at[pl.ds(T * blk, T)], thb)

    def tile_body(i, accs):
        base = jnp.left_shift(i, 3)

        def sb_body(sb, accs):
            accs = list(accs)
            ths = [thb[t, pl.ds(L * sb, L)] for t in range(T)]
            for r in range(8):
                s_ = xa[base + r, pl.ds(L * sb, L)] - ma[base + r, pl.ds(L * sb, L)]
                for t in range(T):
                    accs[t] = accs[t] + jnp.maximum(s_, ths[t])
            return tuple(accs)

        return lax.fori_loop(0, NSB, sb_body, accs)

    accs = lax.fori_loop(off, off + ntiles, tile_body,
                         tuple(zero for _ in range(T)))

    acc = accs[0]
    for t in range(1, T):
        acc = acc + accs[t]
    avec[...] = acc
    tvec[...] = psum * jnp.float32(-C)
    pltpu.sync_copy(avec, out.at[wid])
    pltpu.sync_copy(tvec, out.at[NW + wid])


TC_COLS = 128 * (8 - SC_BLOCKS)


def _tc_body(x_ref, m_ref, tgt_ref, o_ref):
    d = x_ref[...] - m_ref[...]
    j = lax.broadcasted_iota(jnp.int32, (C, 1), 0)
    acc = jnp.float32(0.0)
    gsum = jnp.float32(0.0)
    for t in range(T):
        tg = tgt_ref[t, :][None, :]
        g = jnp.sum(jnp.where(j == tg, d, 0.0), axis=0)
        th = g - 1.0
        acc = acc + jnp.sum(jnp.maximum(d, th[None, :]))
        gsum = gsum + jnp.sum(g)
    o_ref[...] = jnp.full((8, 128), acc - jnp.float32(C) * gsum, jnp.float32)


_tc_call = pl.pallas_call(
    _tc_body,
    grid=(1,),
    in_specs=[
        pl.BlockSpec((C, TC_COLS), lambda i: (0, SC_BLOCKS * 128 // TC_COLS)),
        pl.BlockSpec((C, TC_COLS), lambda i: (0, SC_BLOCKS * 128 // TC_COLS)),
        pl.BlockSpec((T, TC_COLS), lambda i: (0, SC_BLOCKS * 128 // TC_COLS)),
    ],
    out_specs=pl.BlockSpec((8, 128), lambda i: (0, 0)),
    out_shape=jax.ShapeDtypeStruct((8, 128), jnp.float32),
)


def kernel(input_data, target, adaptive_margin):
    xt = input_data.T
    mt = adaptive_margin.T
    tgtT = target.astype(jnp.int32).T
    sc_out = _loss_kernel(xt, mt, tgtT)
    tc_out = _tc_call(xt, mt, tgtT)
    total = jnp.sum(sc_out) + tc_out[0, 0] + jnp.float32((C - 1) * B * T)
    return total / jnp.float32(C)
